# baseline (speedup 1.0000x reference)
import functools

import jax
import jax.numpy as jnp
from jax import lax
from jax.experimental import pallas as pl
from jax.experimental.pallas import tpu as pltpu
from jax.experimental.pallas import tpu_sc as plsc

_NC, _NS, _L = 2, 16, 16
_NW = _NC * _NS
_C = 128
_PACK = 4


def kernel(user, item, user_table, item_table):
    batch = user.shape[0]
    dim = user_table.shape[1]
    assert batch % (_NW * _C) == 0 and dim == 2 * _L
    bpw = batch // _NW
    nch = bpw // _C
    wide = _PACK * dim

    mesh = plsc.VectorSubcoreMesh(
        core_axis_name="c", subcore_axis_name="s",
        num_cores=_NC, num_subcores=_NS,
    )
    cp = pltpu.CompilerParams(needs_layout_passes=False)

    @functools.partial(
        pl.kernel,
        out_type=jax.ShapeDtypeStruct((batch,), jnp.float32),
        mesh=mesh,
        compiler_params=cp,
        scratch_types=[
            pltpu.VMEM((bpw,), jnp.int32),
            pltpu.VMEM((bpw,), jnp.int32),
            pltpu.VMEM((nch, _C), jnp.int32),
            pltpu.VMEM((nch, _C), jnp.int32),
            pltpu.VMEM((2, _C, wide), jnp.float32),
            pltpu.VMEM((2, _C, wide), jnp.float32),
            pltpu.VMEM((bpw,), jnp.float32),
            pltpu.SemaphoreType.DMA,
            pltpu.SemaphoreType.DMA,
        ],
    )
    def sc_kernel(user_hbm, item_hbm, utab_hbm, itab_hbm, out_hbm,
                  uidx_v, iidx_v, ush_v, ish_v, ubuf, ibuf, out_v,
                  sem_u, sem_i):
        wid = lax.axis_index("s") * _NC + lax.axis_index("c")
        base = wid * bpw
        pltpu.sync_copy(user_hbm.at[pl.ds(base, bpw)], uidx_v)
        pltpu.sync_copy(item_hbm.at[pl.ds(base, bpw)], iidx_v)

        for c in range(nch):
            for j in range(_C // _L):
                s = pl.ds(c * _C + j * _L, _L)
                d = pl.ds(j * _L, _L)
                ush_v.at[c][d] = lax.shift_right_logical(uidx_v[s], 2)
                ish_v.at[c][d] = lax.shift_right_logical(iidx_v[s], 2)

        def issue(c):
            bank = c % 2
            return (
                pltpu.async_copy(utab_hbm.at[ush_v.at[c]], ubuf.at[bank],
                                 sem_u),
                pltpu.async_copy(itab_hbm.at[ish_v.at[c]], ibuf.at[bank],
                                 sem_i),
            )

        lane = lax.iota(jnp.int32, _L)
        inflight = issue(0)
        for c in range(nch):
            nxt = issue(c + 1) if c + 1 < nch else None
            inflight[0].wait()
            inflight[1].wait()
            bank = c % 2
            ur, ir = ubuf.at[bank], ibuf.at[bank]

            @pl.loop(0, _C, step=_L)
            def _(r0):
                g = c * _C + r0
                usub = (uidx_v[pl.ds(g, _L)] & (_PACK - 1)) * dim
                isub = (iidx_v[pl.ds(g, _L)] & (_PACK - 1)) * dim
                rows = r0 + lane
                acc = jnp.zeros((_L,), jnp.float32)
                for d in range(dim):
                    u_d = plsc.load_gather(ur, [rows, usub + d])
                    v_d = plsc.load_gather(ir, [rows, isub + d])
                    acc = acc + u_d * v_d
                out_v[pl.ds(g, _L)] = acc

            inflight = nxt

        pltpu.sync_copy(out_v, out_hbm.at[pl.ds(base, bpw)])

    scale = (user[0] // (user_table.shape[0] + 1)).astype(jnp.float32) + 1.0
    ut = user_table.reshape(-1, wide) * scale
    it = item_table.reshape(-1, wide) * scale
    return sc_kernel(user, item, ut, it)


# speedup vs baseline: 2.7964x; 2.7964x over previous
import functools

import jax
import jax.numpy as jnp
from jax import lax
from jax.experimental import pallas as pl
from jax.experimental.pallas import tpu as pltpu
from jax.experimental.pallas import tpu_sc as plsc

_NC, _NS, _L = 2, 16, 16
_NW = _NC * _NS
_W = 128


def kernel(user, item, user_table, item_table):
    batch = user.shape[0]
    nvec, dim = user_table.shape
    assert batch % (_NW * _L) == 0 and dim == 2 * _L
    bpw = batch // _NW
    nt = (nvec // _W) * _W
    ntail = nvec - nt
    cmax = nt - _W

    mesh = plsc.VectorSubcoreMesh(
        core_axis_name="c", subcore_axis_name="s",
        num_cores=_NC, num_subcores=_NS,
    )
    cp = pltpu.CompilerParams(needs_layout_passes=False)

    @functools.partial(
        pl.kernel,
        out_type=jax.ShapeDtypeStruct((batch,), jnp.float32),
        mesh=mesh,
        compiler_params=cp,
        scratch_types=[
            pltpu.VMEM((bpw,), jnp.int32),
            pltpu.VMEM((bpw,), jnp.int32),
            pltpu.VMEM((dim, _W), jnp.float32),
            pltpu.VMEM((dim, _W), jnp.float32),
            pltpu.VMEM((dim, ntail), jnp.float32),
            pltpu.VMEM((dim, ntail), jnp.float32),
            pltpu.VMEM((bpw,), jnp.float32),
            pltpu.SemaphoreType.DMA,
            pltpu.SemaphoreType.DMA,
        ],
    )
    def sc_kernel(user_hbm, item_hbm, utab_hbm, itab_hbm,
                  utail_hbm, itail_hbm, out_hbm,
                  uidx_v, iidx_v, upan, ipan,
                  utail_v, itail_v, out_v, sem_u, sem_i):
        sid = lax.axis_index("s")
        wid = sid * _NC + lax.axis_index("c")
        base = wid * bpw
        pltpu.sync_copy(user_hbm.at[pl.ds(base, bpw)], uidx_v)
        pltpu.sync_copy(item_hbm.at[pl.ds(base, bpw)], iidx_v)
        pltpu.sync_copy(utail_hbm, utail_v)
        pltpu.sync_copy(itail_hbm, itail_v)

        lane = lax.iota(jnp.int32, _L)
        last_lane = lane == (_L - 1)

        def start(val):
            return pl.multiple_of(
                jnp.minimum((val // _W) * _W, cmax), _W)

        def column(panel, tail, val):
            lu = jnp.full((_L,), jnp.minimum(val - start(val), _W - 1),
                          jnp.int32)
            tl = jnp.full((_L,), jnp.clip(val - nt, 0, ntail - 1),
                          jnp.int32)
            m0 = plsc.load_gather(panel, [lane, lu])
            m1 = plsc.load_gather(panel, [lane + _L, lu])
            t0 = plsc.load_gather(tail, [lane, tl])
            t1 = plsc.load_gather(tail, [lane + _L, tl])
            in_tail = jnp.full((_L,), val >= nt)
            return (jnp.where(in_tail, t0, m0), jnp.where(in_tail, t1, m1))

        @pl.loop(0, bpw, step=_L)
        def _(n0):
            uchunk = uidx_v[pl.ds(n0, _L)]
            ichunk = iidx_v[pl.ds(n0, _L)]
            for k in range(_L):
                uval = jax.lax.reduce_sum(
                    jnp.where(lane == k, uchunk, 0), axes=(0,))
                ival = jax.lax.reduce_sum(
                    jnp.where(lane == k, ichunk, 0), axes=(0,))
                cp_u = pltpu.async_copy(
                    utab_hbm.at[:, pl.ds(start(uval), _W)], upan, sem_u)
                cp_i = pltpu.async_copy(
                    itab_hbm.at[:, pl.ds(start(ival), _W)], ipan, sem_i)
                cp_u.wait()
                cp_i.wait()
                u0, u1 = column(upan, utail_v, uval)
                v0, v1 = column(ipan, itail_v, ival)
                s = u0 * v0 + u1 * v1
                tot = plsc.cumsum(s)
                plsc.store_scatter(out_v,
                                   [jnp.full((_L,), n0 + k, jnp.int32)],
                                   tot, mask=last_lane)

        pltpu.sync_copy(out_v, out_hbm.at[pl.ds(base, bpw)])

    utab = user_table.T
    itab = item_table.T
    return sc_kernel(user, item, utab, itab,
                     utab[:, nt:], itab[:, nt:])


# speedup vs baseline: 6.2431x; 2.2325x over previous
import functools

import jax
import jax.numpy as jnp
from jax import lax
from jax.experimental import pallas as pl
from jax.experimental.pallas import tpu as pltpu
from jax.experimental.pallas import tpu_sc as plsc

_NC, _NS, _L = 2, 16, 16
_NW = _NC * _NS
_W = 128
_R = 8


def kernel(user, item, user_table, item_table):
    batch = user.shape[0]
    nvec, dim = user_table.shape
    assert batch % (_NW * _L) == 0 and dim == 2 * _L
    bpw = batch // _NW
    nt = (nvec // _W) * _W
    ntail = nvec - nt
    cmax = nt - _W

    mesh = plsc.VectorSubcoreMesh(
        core_axis_name="c", subcore_axis_name="s",
        num_cores=_NC, num_subcores=_NS,
    )
    cp = pltpu.CompilerParams(needs_layout_passes=False)

    @functools.partial(
        pl.kernel,
        out_type=jax.ShapeDtypeStruct((batch,), jnp.float32),
        mesh=mesh,
        compiler_params=cp,
        scratch_types=[
            pltpu.VMEM((bpw,), jnp.int32),
            pltpu.VMEM((bpw,), jnp.int32),
            pltpu.VMEM((_R, dim, _W), jnp.float32),
            pltpu.VMEM((_R, dim, _W), jnp.float32),
            pltpu.VMEM((dim, ntail), jnp.float32),
            pltpu.VMEM((dim, ntail), jnp.float32),
            pltpu.VMEM((bpw,), jnp.float32),
            [pltpu.SemaphoreType.DMA] * _R,
            [pltpu.SemaphoreType.DMA] * _R,
        ],
    )
    def sc_kernel(user_hbm, item_hbm, utab_hbm, itab_hbm,
                  utail_hbm, itail_hbm, out_hbm,
                  uidx_v, iidx_v, upan, ipan,
                  utail_v, itail_v, out_v, sems_u, sems_i):
        sid = lax.axis_index("s")
        wid = sid * _NC + lax.axis_index("c")
        base = wid * bpw
        pltpu.sync_copy(user_hbm.at[pl.ds(base, bpw)], uidx_v)
        pltpu.sync_copy(item_hbm.at[pl.ds(base, bpw)], iidx_v)
        pltpu.sync_copy(utail_hbm, utail_v)
        pltpu.sync_copy(itail_hbm, itail_v)

        lane = lax.iota(jnp.int32, _L)
        last_lane = lane == (_L - 1)

        def start(val):
            return pl.multiple_of(
                jnp.minimum((val // _W) * _W, cmax), _W)

        def column(panel, tail, val):
            lu = jnp.full((_L,), jnp.minimum(val - start(val), _W - 1),
                          jnp.int32)
            tl = jnp.full((_L,), jnp.clip(val - nt, 0, ntail - 1),
                          jnp.int32)
            m0 = plsc.load_gather(panel, [lane, lu])
            m1 = plsc.load_gather(panel, [lane + _L, lu])
            t0 = plsc.load_gather(tail, [lane, tl])
            t1 = plsc.load_gather(tail, [lane + _L, tl])
            in_tail = jnp.full((_L,), val >= nt)
            return (jnp.where(in_tail, t0, m0), jnp.where(in_tail, t1, m1))

        def pick(chunk, k):
            return jax.lax.reduce_sum(
                jnp.where(lane == k, chunk, 0), axes=(0,))

        def issue(slot, uval, ival):
            pltpu.async_copy(utab_hbm.at[:, pl.ds(start(uval), _W)],
                             upan.at[slot], sems_u[slot])
            pltpu.async_copy(itab_hbm.at[:, pl.ds(start(ival), _W)],
                             ipan.at[slot], sems_i[slot])

        chunk0_u = uidx_v[pl.ds(0, _L)]
        chunk0_i = iidx_v[pl.ds(0, _L)]
        for k in range(_R):
            issue(k, pick(chunk0_u, k), pick(chunk0_i, k))

        @pl.loop(0, bpw, step=_L)
        def _(n0):
            uchunk = uidx_v[pl.ds(n0, _L)]
            ichunk = iidx_v[pl.ds(n0, _L)]
            nn = jnp.minimum(n0 + _L, bpw - _L)
            unext = uidx_v[pl.ds(nn, _L)]
            inext = iidx_v[pl.ds(nn, _L)]
            for k in range(_L):
                slot = k % _R
                pltpu.make_async_copy(utab_hbm.at[:, pl.ds(0, _W)],
                                      upan.at[slot], sems_u[slot]).wait()
                pltpu.make_async_copy(itab_hbm.at[:, pl.ds(0, _W)],
                                      ipan.at[slot], sems_i[slot]).wait()
                uval = pick(uchunk, k)
                ival = pick(ichunk, k)
                u0, u1 = column(upan.at[slot], utail_v, uval)
                v0, v1 = column(ipan.at[slot], itail_v, ival)
                s = u0 * v0 + u1 * v1
                tot = plsc.cumsum(s)
                plsc.store_scatter(out_v,
                                   [jnp.full((_L,), n0 + k, jnp.int32)],
                                   tot, mask=last_lane)

                @pl.when(n0 + k + _R < bpw)
                def _():
                    if k + _R < _L:
                        issue(slot, pick(uchunk, k + _R),
                              pick(ichunk, k + _R))
                    else:
                        issue(slot, pick(unext, k + _R - _L),
                              pick(inext, k + _R - _L))

        pltpu.sync_copy(out_v, out_hbm.at[pl.ds(base, bpw)])

    utab = user_table.T
    itab = item_table.T
    return sc_kernel(user, item, utab, itab,
                     utab[:, nt:], itab[:, nt:])
